# baseline (speedup 1.0000x reference)
import functools

import jax
import jax.numpy as jnp
from jax import lax
from jax.experimental import pallas as pl
from jax.experimental.pallas import tpu as pltpu
from jax.experimental.pallas import tpu_sc as plsc

NC, NS, L = 2, 16, 16
NW = NC * NS
TILE = 128
CHUNK = 16
NBUF = 3


def kernel(decoder_states, encoder_states, step):
    batch, enc, _ = encoder_states.shape
    _, dec, _ = decoder_states.shape

    value = (jnp.asarray(step, jnp.float32) + 2.0) * jnp.ones((L,), jnp.float32)

    rows_total = batch * dec
    rpw = rows_total // NW
    n_chunks = rpw // CHUNK

    mesh = plsc.VectorSubcoreMesh(core_axis_name="c", subcore_axis_name="s")

    @functools.partial(
        pl.kernel,
        mesh=mesh,
        out_type=jax.ShapeDtypeStruct((batch, dec, enc), jnp.float32),
        scratch_types=[pltpu.VMEM((L,), jnp.float32)]
        + [pltpu.VMEM((CHUNK, enc), jnp.float32)] * NBUF
        + [pltpu.SemaphoreType.DMA] * (NBUF + 1),
    )
    def sc_fill(val_hbm, out_hbm, val_v, *rest):
        bufs, sems, vsem = rest[:NBUF], rest[NBUF : 2 * NBUF], rest[2 * NBUF]
        wid = lax.axis_index("s") * NC + lax.axis_index("c")
        row_base = wid * rpw
        b = row_base // dec
        drow_base = row_base % dec
        drow_base = pl.multiple_of(drow_base, TILE)

        vcopy = pltpu.async_copy(val_hbm, val_v, vsem)
        ninf = jnp.full((L,), -jnp.inf, jnp.float32)
        iota = lax.broadcasted_iota(jnp.int32, (L,), 0)

        def fill_ninf(buf):
            def body_r(r, carry):
                def body(j, carry2):
                    buf[r, pl.ds(j * L, L)] = ninf
                    return carry2

                return lax.fori_loop(0, enc // L, body, carry)

            lax.fori_loop(0, CHUNK, body_r, 0)

        def diag_window(c):
            return pl.multiple_of(drow_base + c * CHUNK, CHUNK)

        def poke(buf, c, vec_fn):
            cw = diag_window(c)
            for i in range(CHUNK):
                buf[i, pl.ds(cw, L)] = vec_fn(i)

        def blk_dst(c):
            r0 = drow_base + c * CHUNK
            return out_hbm.at[b, pl.ds(r0, CHUNK), :]

        copies = [None] * NBUF
        val_vec = None
        for c in range(NBUF):
            fill_ninf(bufs[c])
            if val_vec is None:
                vcopy.wait()
                val_vec = val_v[...]
                diag_vec = lambda i: jnp.where(iota == i, val_vec, ninf)
            poke(bufs[c], c, diag_vec)
            copies[c] = pltpu.async_copy(bufs[c], blk_dst(c), sems[c])

        for c in range(NBUF, n_chunks):
            s = c % NBUF
            buf = bufs[s]
            copies[s].wait()
            poke(buf, c - NBUF, lambda i: ninf)
            poke(buf, c, diag_vec)
            copies[s] = pltpu.async_copy(buf, blk_dst(c), sems[s])
        for s in range(NBUF):
            copies[s].wait()

    return sc_fill(value)


# speedup vs baseline: 1.2754x; 1.2754x over previous
import functools

import jax
import jax.numpy as jnp
from jax import lax
from jax.experimental import pallas as pl
from jax.experimental.pallas import tpu as pltpu
from jax.experimental.pallas import tpu_sc as plsc

NC, NS, L = 2, 16, 16
NW = NC * NS
TILE = 128
CHUNK = 16
DEPTH = 16


def kernel(decoder_states, encoder_states, step):
    batch, enc, _ = encoder_states.shape
    _, dec, _ = decoder_states.shape

    value = (jnp.asarray(step, jnp.float32) + 2.0) * jnp.ones((L,), jnp.float32)

    rows_total = batch * dec
    rpw = rows_total // NW
    n_chunks = rpw // CHUNK
    n_pat = TILE // CHUNK

    mesh = plsc.VectorSubcoreMesh(core_axis_name="c", subcore_axis_name="s")

    @functools.partial(
        pl.kernel,
        mesh=mesh,
        out_type=jax.ShapeDtypeStruct((batch, dec, enc), jnp.float32),
        scratch_types=[
            pltpu.VMEM((L,), jnp.float32),
            pltpu.VMEM((CHUNK, enc), jnp.float32),
            pltpu.VMEM((CHUNK, n_pat * TILE), jnp.float32),
        ]
        + [pltpu.SemaphoreType.DMA] * (DEPTH + 2),
    )
    def sc_fill(val_hbm, out_hbm, val_v, blk_v, pat_v, *sems_all):
        sems, psem, vsem = sems_all[:DEPTH], sems_all[DEPTH], sems_all[DEPTH + 1]
        wid = lax.axis_index("s") * NC + lax.axis_index("c")
        row_base = wid * rpw
        b = row_base // dec
        drow_base = row_base % dec
        drow_base = pl.multiple_of(drow_base, TILE)

        vcopy = pltpu.async_copy(val_hbm, val_v, vsem)
        ninf = jnp.full((L,), -jnp.inf, jnp.float32)
        iota = lax.broadcasted_iota(jnp.int32, (L,), 0)

        def fill_ninf(buf, width):
            def body_r(r, carry):
                def body(j, carry2):
                    buf[r, pl.ds(j * L, L)] = ninf
                    return carry2

                return lax.fori_loop(0, width // L, body, carry)

            lax.fori_loop(0, CHUNK, body_r, 0)

        def blk_dst(c):
            r0 = drow_base + c * CHUNK
            return out_hbm.at[b, pl.ds(r0, CHUNK), :]

        def patch_dst(c):
            r0 = drow_base + c * CHUNK
            ct = drow_base + (c // n_pat) * TILE
            return out_hbm.at[b, pl.ds(r0, CHUNK), pl.ds(pl.multiple_of(ct, TILE), TILE)]

        blk_copies = [None] * DEPTH
        patch_copies = []

        fill_ninf(blk_v, enc)
        for c in range(DEPTH):
            blk_copies[c % DEPTH] = pltpu.async_copy(blk_v, blk_dst(c), sems[c % DEPTH])

        fill_ninf(pat_v, n_pat * TILE)
        vcopy.wait()
        val_vec = val_v[...]
        for p in range(n_pat):
            for i in range(CHUNK):
                w = p * (TILE + CHUNK) + (i // L) * L
                pat_v[i, pl.ds(w, L)] = jnp.where(iota == i % L, val_vec, ninf)

        for c in range(DEPTH, n_chunks):
            s = c % DEPTH
            blk_copies[s].wait()
            patch_copies.append(
                pltpu.async_copy(
                    pat_v.at[:, pl.ds(((c - DEPTH) % n_pat) * TILE, TILE)],
                    patch_dst(c - DEPTH),
                    psem,
                )
            )
            blk_copies[s] = pltpu.async_copy(blk_v, blk_dst(c), sems[s])
        for c in range(n_chunks - DEPTH, n_chunks):
            blk_copies[c % DEPTH].wait()
            patch_copies.append(
                pltpu.async_copy(
                    pat_v.at[:, pl.ds((c % n_pat) * TILE, TILE)],
                    patch_dst(c),
                    psem,
                )
            )
        for h in patch_copies:
            h.wait()

    return sc_fill(value)
